# Initial kernel scaffold; baseline (speedup 1.0000x reference)
import jax
import jax.numpy as jnp
from jax.experimental import pallas as pl


def kernel(x, edge_index, W0, a_src0, a_dst0, b0, W1, a_src1, a_dst1, b1, Wc, bc):
    raise NotImplementedError("write your pallas kernel here")


# speedup vs baseline: 17.2568x; 17.2568x over previous
import functools

import jax
import jax.numpy as jnp
from jax import lax
from jax.experimental import pallas as pl
from jax.experimental.pallas import tpu as pltpu
from jax.experimental.pallas import tpu_sc as plsc

N = 10000
NP = 10240
CH = 128
OUT = 64
E_RAW = 320000
E_TOT = E_RAW + N
NC = 2
NS = 16
NW = NC * NS
BLK = 128
NBLK = 81
EPW = NBLK * BLK
EP = NW * EPW
PAD_IDX = N
RPT = NP // NS


def _tc_pre_body(x_ref, w_ref, asr_ref, adr_ref, h_ref, as_ref, ad_ref, m_ref):
    h = jnp.dot(x_ref[...], w_ref[...], preferred_element_type=jnp.float32)
    h_ref[...] = h
    a_s = jnp.sum(h * asr_ref[...][None, :], axis=1)
    a_d = jnp.sum(h * adr_ref[...][None, :], axis=1)
    as_ref[...] = a_s
    ad_ref[...] = a_d
    t = jnp.max(a_s) + a_d
    m_ref[...] = jnp.where(t > 0, t, 0.2 * t)


def _tc_mid_body(op_ref, dp_ref, b_ref, w_ref, asr_ref, adr_ref,
                 h_ref, as_ref, ad_ref, m_ref):
    den = dp_ref[0, :] + dp_ref[1, :] + 1e-16
    o = (op_ref[0] + op_ref[1]) / den[:, None] + b_ref[...][None, :]
    o = jnp.maximum(o, 0.0)
    h = jnp.dot(o, w_ref[...], preferred_element_type=jnp.float32)
    h_ref[...] = h
    a_s = jnp.sum(h * asr_ref[...][None, :], axis=1)
    a_d = jnp.sum(h * adr_ref[...][None, :], axis=1)
    as_ref[...] = a_s
    ad_ref[...] = a_d
    t = jnp.max(a_s) + a_d
    m_ref[...] = jnp.where(t > 0, t, 0.2 * t)


def _tc_fin_body(op_ref, dp_ref, b_ref, wc_ref, bc_ref, y_ref):
    den = dp_ref[0, :] + dp_ref[1, :] + 1e-16
    o = (op_ref[0] + op_ref[1]) / den[:, None] + b_ref[...][None, :]
    o = jnp.maximum(o, 0.0)
    y_ref[...] = (jnp.dot(o, wc_ref[...], preferred_element_type=jnp.float32)
                  + bc_ref[...][None, :])


_f32 = jnp.float32

_tc_pre = pl.pallas_call(
    _tc_pre_body,
    out_shape=(jax.ShapeDtypeStruct((NP, CH), _f32),
               jax.ShapeDtypeStruct((NP,), _f32),
               jax.ShapeDtypeStruct((NP,), _f32),
               jax.ShapeDtypeStruct((NP,), _f32)),
)

_tc_mid = pl.pallas_call(
    _tc_mid_body,
    out_shape=(jax.ShapeDtypeStruct((NP, CH), _f32),
               jax.ShapeDtypeStruct((NP,), _f32),
               jax.ShapeDtypeStruct((NP,), _f32),
               jax.ShapeDtypeStruct((NP,), _f32)),
)

_tc_fin = pl.pallas_call(
    _tc_fin_body,
    out_shape=jax.ShapeDtypeStruct((NP, OUT), _f32),
)


def _sc_layer_body(h_hbm, as_hbm, ad_hbm, m_hbm, src_hbm, dst_hbm,
                   outp_hbm, denp_hbm,
                   ase_v, ade_v, me_v, src_v, dst_v, w_v, rows_v,
                   out_sp, den_sp):
    c = lax.axis_index("c")
    s = lax.axis_index("s")
    wid = c * NS + s

    pltpu.sync_copy(src_hbm.at[wid], src_v)
    pltpu.sync_copy(dst_hbm.at[wid], dst_v)

    zeros16 = jnp.zeros((16,), _f32)

    @pl.loop(0, BLK)
    def _zero_rows(i):
        for cg in range(CH // 16):
            rows_v[i, pl.ds(cg * 16, 16)] = zeros16

    for g in range(BLK // 16):
        w_v[pl.ds(g * 16, 16)] = zeros16
    for k in range(RPT // BLK):
        pltpu.sync_copy(rows_v, out_sp.at[pl.ds(s * RPT + k * BLK, BLK)])
        pltpu.sync_copy(w_v, den_sp.at[pl.ds(s * RPT + k * BLK, BLK)])
    plsc.subcore_barrier()

    @pl.loop(0, NBLK)
    def _block(j):
        pltpu.sync_copy(h_hbm.at[src_v.at[j]], rows_v)
        pltpu.sync_copy(as_hbm.at[src_v.at[j]], ase_v)
        pltpu.sync_copy(ad_hbm.at[dst_v.at[j]], ade_v)
        pltpu.sync_copy(m_hbm.at[dst_v.at[j]], me_v)
        for g in range(BLK // 16):
            a_s = ase_v[pl.ds(g * 16, 16)]
            a_d = ade_v[pl.ds(g * 16, 16)]
            mm = me_v[pl.ds(g * 16, 16)]
            t = a_s + a_d
            e = jnp.where(t > 0, t, 0.2 * t)
            w_v[pl.ds(g * 16, 16)] = jnp.exp(e - mm)

        @pl.loop(0, BLK)
        def _scale(i):
            bidx = jnp.zeros((16,), jnp.int32) + i
            a16 = plsc.load_gather(w_v, [bidx])
            for cg in range(CH // 16):
                rows_v[i, pl.ds(cg * 16, 16)] = rows_v[i, pl.ds(cg * 16, 16)] * a16

        pltpu.sync_copy(w_v, den_sp.at[dst_v.at[j]], add=True)
        pltpu.sync_copy(rows_v, out_sp.at[dst_v.at[j]], add=True)

    plsc.subcore_barrier()
    for k in range(RPT // BLK):
        pltpu.sync_copy(out_sp.at[pl.ds(s * RPT + k * BLK, BLK)],
                        outp_hbm.at[c, pl.ds(s * RPT + k * BLK, BLK)])
    pltpu.sync_copy(den_sp.at[pl.ds(s * RPT, RPT)],
                    denp_hbm.at[c, pl.ds(s * RPT, RPT)])


@functools.cache
def _make_sc_layer():
    mesh = plsc.VectorSubcoreMesh(core_axis_name="c", subcore_axis_name="s",
                                  num_cores=NC, num_subcores=NS)
    return pl.kernel(
        _sc_layer_body,
        out_type=(jax.ShapeDtypeStruct((NC, NP, CH), _f32),
                  jax.ShapeDtypeStruct((NC, NP), _f32)),
        mesh=mesh,
        compiler_params=pltpu.CompilerParams(needs_layout_passes=False),
        scratch_types=[
            pltpu.VMEM((BLK,), _f32),
            pltpu.VMEM((BLK,), _f32),
            pltpu.VMEM((BLK,), _f32),
            pltpu.VMEM((NBLK, BLK), jnp.int32),
            pltpu.VMEM((NBLK, BLK), jnp.int32),
            pltpu.VMEM((BLK,), _f32),
            pltpu.VMEM((BLK, CH), _f32),
            pltpu.VMEM_SHARED((NP, CH), _f32),
            pltpu.VMEM_SHARED((NP,), _f32),
        ],
    )


def kernel(x, edge_index, W0, a_src0, a_dst0, b0, W1, a_src1, a_dst1, b1,
           Wc, bc):
    ei = edge_index.astype(jnp.int32)
    ar = jnp.arange(N, dtype=jnp.int32)
    pad = jnp.full((EP - E_TOT,), PAD_IDX, jnp.int32)
    src = jnp.concatenate([ei[0], ar, pad]).reshape(NW, NBLK, BLK)
    dst = jnp.concatenate([ei[1], ar, pad]).reshape(NW, NBLK, BLK)
    xp = jnp.pad(x, ((0, NP - N), (0, 0)))

    sc_layer = _make_sc_layer()
    h0, as0, ad0, m0 = _tc_pre(xp, W0, a_src0, a_dst0)
    op0, dp0 = sc_layer(h0, as0, ad0, m0, src, dst)
    h1, as1, ad1, m1 = _tc_mid(op0, dp0, b0, W1, a_src1, a_dst1)
    op1, dp1 = sc_layer(h1, as1, ad1, m1, src, dst)
    y = _tc_fin(op1, dp1, b1, Wc, bc)
    return y[:N]


# speedup vs baseline: 32.0811x; 1.8590x over previous
import functools

import jax
import jax.numpy as jnp
from jax import lax
from jax.experimental import pallas as pl
from jax.experimental.pallas import tpu as pltpu
from jax.experimental.pallas import tpu_sc as plsc

N = 10000
NP = 10240
CH = 128
OUT = 64
E_RAW = 320000
E_TOT = E_RAW + N
NC = 2
NS = 16
NW = NC * NS
BLK = 112
NBLK = 93
EPW = NBLK * BLK
EP = NW * EPW
PAD_IDX = N
RPT = NP // NS
NRING = 3
IRING = 8


def _tc_pre_body(x_ref, w_ref, asr_ref, adr_ref, h_ref, as_ref, ad_ref, m_ref):
    h = jnp.dot(x_ref[...], w_ref[...], preferred_element_type=jnp.float32)
    h_ref[...] = h
    a_s = jnp.sum(h * asr_ref[...][None, :], axis=1)
    a_d = jnp.sum(h * adr_ref[...][None, :], axis=1)
    as_ref[...] = a_s
    ad_ref[...] = a_d
    t = jnp.max(a_s) + a_d
    m_ref[...] = jnp.where(t > 0, t, 0.2 * t)


def _tc_mid_body(op_ref, dp_ref, b_ref, w_ref, asr_ref, adr_ref,
                 h_ref, as_ref, ad_ref, m_ref):
    den = dp_ref[0, :] + dp_ref[1, :] + 1e-16
    o = (op_ref[0] + op_ref[1]) / den[:, None] + b_ref[...][None, :]
    o = jnp.maximum(o, 0.0)
    h = jnp.dot(o, w_ref[...], preferred_element_type=jnp.float32)
    h_ref[...] = h
    a_s = jnp.sum(h * asr_ref[...][None, :], axis=1)
    a_d = jnp.sum(h * adr_ref[...][None, :], axis=1)
    as_ref[...] = a_s
    ad_ref[...] = a_d
    t = jnp.max(a_s) + a_d
    m_ref[...] = jnp.where(t > 0, t, 0.2 * t)


def _tc_fin_body(op_ref, dp_ref, b_ref, wc_ref, bc_ref, y_ref):
    den = dp_ref[0, :] + dp_ref[1, :] + 1e-16
    o = (op_ref[0] + op_ref[1]) / den[:, None] + b_ref[...][None, :]
    o = jnp.maximum(o, 0.0)
    y_ref[...] = (jnp.dot(o, wc_ref[...], preferred_element_type=jnp.float32)
                  + bc_ref[...][None, :])


_f32 = jnp.float32

_tc_pre = pl.pallas_call(
    _tc_pre_body,
    out_shape=(jax.ShapeDtypeStruct((NP, CH), _f32),
               jax.ShapeDtypeStruct((NP,), _f32),
               jax.ShapeDtypeStruct((NP,), _f32),
               jax.ShapeDtypeStruct((NP,), _f32)),
)

_tc_mid = pl.pallas_call(
    _tc_mid_body,
    out_shape=(jax.ShapeDtypeStruct((NP, CH), _f32),
               jax.ShapeDtypeStruct((NP,), _f32),
               jax.ShapeDtypeStruct((NP,), _f32),
               jax.ShapeDtypeStruct((NP,), _f32)),
)

_tc_fin = pl.pallas_call(
    _tc_fin_body,
    out_shape=jax.ShapeDtypeStruct((NP, OUT), _f32),
)


def _sc_layer_body(h_hbm, as_hbm, ad_hbm, m_hbm, idx_hbm,
                   outp_hbm, denp_hbm,
                   rows_b, w_b, ase_b, ade_b, me_b, idx_b,
                   rows_sem, sc_sem, idx_sem, scat_sem,
                   out_sp, den_sp):
    c = lax.axis_index("c")
    s = lax.axis_index("s")
    wid = c * NS + s

    def _idx_copy(k):
        return pltpu.make_async_copy(
            idx_hbm.at[wid, k], idx_b.at[lax.rem(k, IRING)], idx_sem)

    def _gather_descs(k, r):
        k8 = lax.rem(k, IRING)
        return [
            pltpu.make_async_copy(h_hbm.at[idx_b.at[k8, 0]], rows_b.at[r],
                                  rows_sem),
            pltpu.make_async_copy(as_hbm.at[idx_b.at[k8, 0]], ase_b.at[r],
                                  sc_sem),
            pltpu.make_async_copy(ad_hbm.at[idx_b.at[k8, 1]], ade_b.at[r],
                                  sc_sem),
            pltpu.make_async_copy(m_hbm.at[idx_b.at[k8, 1]], me_b.at[r],
                                  sc_sem),
        ]

    def _issue_scatters(k, r):
        k8 = lax.rem(k, IRING)
        pltpu.async_copy(w_b.at[r], den_sp.at[idx_b.at[k8, 1]], scat_sem,
                         add=True)
        pltpu.async_copy(rows_b.at[r], out_sp.at[idx_b.at[k8, 1]], scat_sem,
                         add=True)

    def _wait_scatters(k, r):
        k8 = lax.rem(k, IRING)
        pltpu.make_async_copy(w_b.at[r], den_sp.at[idx_b.at[k8, 1]],
                              scat_sem).wait()
        pltpu.make_async_copy(rows_b.at[r], out_sp.at[idx_b.at[k8, 1]],
                              scat_sem).wait()

    zeros16 = jnp.zeros((16,), _f32)

    @pl.loop(0, BLK)
    def _zero_rows(i):
        for cg in range(CH // 16):
            rows_b[0, i, pl.ds(cg * 16, 16)] = zeros16

    for g in range(BLK // 16):
        w_b[0, pl.ds(g * 16, 16)] = zeros16
    for t in range(RPT // BLK):
        pltpu.sync_copy(rows_b.at[0], out_sp.at[pl.ds(s * RPT + t * BLK, BLK)])
        pltpu.sync_copy(w_b.at[0], den_sp.at[pl.ds(s * RPT + t * BLK, BLK)])
    _rem = RPT - (RPT // BLK) * BLK
    if _rem:
        pltpu.sync_copy(rows_b.at[0].at[pl.ds(0, _rem)],
                        out_sp.at[pl.ds(s * RPT + (RPT // BLK) * BLK, _rem)])
        pltpu.sync_copy(w_b.at[0].at[pl.ds(0, _rem)],
                        den_sp.at[pl.ds(s * RPT + (RPT // BLK) * BLK, _rem)])
    plsc.subcore_barrier()

    def _step(k, r):
        for d in _gather_descs(k, r):
            d.wait()
        for g in range(BLK // 16):
            a_s = ase_b[r, pl.ds(g * 16, 16)]
            a_d = ade_b[r, pl.ds(g * 16, 16)]
            mm = me_b[r, pl.ds(g * 16, 16)]
            t = a_s + a_d
            e = jnp.where(t > 0, t, 0.2 * t)
            w_b[r, pl.ds(g * 16, 16)] = jnp.exp(e - mm)

        @pl.loop(0, BLK)
        def _scale(i):
            bidx = jnp.zeros((16,), jnp.int32) + i
            a16 = plsc.load_gather(w_b.at[r], [bidx])
            for cg in range(CH // 16):
                rows_b[r, i, pl.ds(cg * 16, 16)] = (
                    rows_b[r, i, pl.ds(cg * 16, 16)] * a16)

        _issue_scatters(k, r)

        @pl.when(k >= 1)
        def _drain():
            _wait_scatters(k - 1, (r - 1) % NRING)

        @pl.when(k + 2 < NBLK)
        def _prefetch():
            _idx_copy(k + 2).wait()
            for d in _gather_descs(k + 2, (r + 2) % NRING):
                d.start()

        @pl.when(k + 4 < NBLK)
        def _prefetch_idx():
            _idx_copy(k + 4).start()

    for k in range(4):
        _idx_copy(k).start()
    for k in range(2):
        _idx_copy(k).wait()
        for d in _gather_descs(k, k):
            d.start()

    @pl.loop(0, NBLK // NRING)
    def _outer(t):
        for b in range(NRING):
            _step(t * NRING + b, b)

    _wait_scatters(NBLK - 1, (NBLK - 1) % NRING)

    plsc.subcore_barrier()
    pltpu.sync_copy(out_sp.at[pl.ds(s * RPT, RPT)],
                    outp_hbm.at[c, pl.ds(s * RPT, RPT)])
    pltpu.sync_copy(den_sp.at[pl.ds(s * RPT, RPT)],
                    denp_hbm.at[c, pl.ds(s * RPT, RPT)])


@functools.cache
def _make_sc_layer():
    mesh = plsc.VectorSubcoreMesh(core_axis_name="c", subcore_axis_name="s",
                                  num_cores=NC, num_subcores=NS)
    return pl.kernel(
        _sc_layer_body,
        out_type=(jax.ShapeDtypeStruct((NC, NP, CH), _f32),
                  jax.ShapeDtypeStruct((NC, NP), _f32)),
        mesh=mesh,
        compiler_params=pltpu.CompilerParams(needs_layout_passes=False),
        scratch_types=[
            pltpu.VMEM((NRING, BLK, CH), _f32),
            pltpu.VMEM((NRING, BLK), _f32),
            pltpu.VMEM((NRING, BLK), _f32),
            pltpu.VMEM((NRING, BLK), _f32),
            pltpu.VMEM((NRING, BLK), _f32),
            pltpu.VMEM((IRING, 2, BLK), jnp.int32),
            pltpu.SemaphoreType.DMA,
            pltpu.SemaphoreType.DMA,
            pltpu.SemaphoreType.DMA,
            pltpu.SemaphoreType.DMA,
            pltpu.VMEM_SHARED((NP, CH), _f32),
            pltpu.VMEM_SHARED((NP,), _f32),
        ],
    )


def kernel(x, edge_index, W0, a_src0, a_dst0, b0, W1, a_src1, a_dst1, b1,
           Wc, bc):
    ei = edge_index.astype(jnp.int32)
    ar = jnp.arange(N, dtype=jnp.int32)
    pad = jnp.full((EP - E_TOT,), PAD_IDX, jnp.int32)
    src = jnp.concatenate([ei[0], ar, pad]).reshape(NW, NBLK, BLK)
    dst = jnp.concatenate([ei[1], ar, pad]).reshape(NW, NBLK, BLK)
    idx = jnp.stack([src, dst], axis=2)
    xp = jnp.pad(x, ((0, NP - N), (0, 0)))

    sc_layer = _make_sc_layer()
    h0, as0, ad0, m0 = _tc_pre(xp, W0, a_src0, a_dst0)
    op0, dp0 = sc_layer(h0, as0, ad0, m0, idx)
    h1, as1, ad1, m1 = _tc_mid(op0, dp0, b0, W1, a_src1, a_dst1)
    op1, dp1 = sc_layer(h1, as1, ad1, m1, idx)
    y = _tc_fin(op1, dp1, b1, Wc, bc)
    return y[:N]


# speedup vs baseline: 32.5985x; 1.0161x over previous
import functools

import jax
import jax.numpy as jnp
from jax import lax
from jax.experimental import pallas as pl
from jax.experimental.pallas import tpu as pltpu
from jax.experimental.pallas import tpu_sc as plsc

N = 10000
NP = 10240
CH = 128
OUT = 64
E_RAW = 320000
E_TOT = E_RAW + N
NC = 2
NS = 16
NW = NC * NS
BLK = 112
NBLK = 93
EPW = NBLK * BLK
EP = NW * EPW
PAD_IDX = N
RPT = NP // NS
NRING = 3
IRING = 8


def _tc_pre_body(x_ref, w_ref, asr_ref, adr_ref, h_ref, as_ref, ad_ref, m_ref):
    h = jnp.dot(x_ref[...], w_ref[...], preferred_element_type=jnp.float32)
    h_ref[...] = h
    a_s = jnp.sum(h * asr_ref[...][None, :], axis=1)
    a_d = jnp.sum(h * adr_ref[...][None, :], axis=1)
    as_ref[...] = a_s
    ad_ref[...] = a_d
    t = jnp.max(a_s) + a_d
    m_ref[...] = jnp.where(t > 0, t, 0.2 * t)


def _tc_mid_body(op_ref, dp_ref, b_ref, w_ref, asr_ref, adr_ref,
                 h_ref, as_ref, ad_ref, m_ref):
    den = dp_ref[0, :] + dp_ref[1, :] + 1e-16
    o = (op_ref[0] + op_ref[1]) / den[:, None] + b_ref[...][None, :]
    o = jnp.maximum(o, 0.0)
    h = jnp.dot(o, w_ref[...], preferred_element_type=jnp.float32)
    h_ref[...] = h
    a_s = jnp.sum(h * asr_ref[...][None, :], axis=1)
    a_d = jnp.sum(h * adr_ref[...][None, :], axis=1)
    as_ref[...] = a_s
    ad_ref[...] = a_d
    t = jnp.max(a_s) + a_d
    m_ref[...] = jnp.where(t > 0, t, 0.2 * t)


def _tc_fin_body(op_ref, dp_ref, b_ref, wc_ref, bc_ref, y_ref):
    den = dp_ref[0, :] + dp_ref[1, :] + 1e-16
    o = (op_ref[0] + op_ref[1]) / den[:, None] + b_ref[...][None, :]
    o = jnp.maximum(o, 0.0)
    y_ref[...] = (jnp.dot(o, wc_ref[...], preferred_element_type=jnp.float32)
                  + bc_ref[...][None, :])


_f32 = jnp.float32

_tc_pre = pl.pallas_call(
    _tc_pre_body,
    out_shape=(jax.ShapeDtypeStruct((NP, CH), _f32),
               jax.ShapeDtypeStruct((NP,), _f32),
               jax.ShapeDtypeStruct((NP,), _f32),
               jax.ShapeDtypeStruct((NP,), _f32)),
)

_tc_mid = pl.pallas_call(
    _tc_mid_body,
    out_shape=(jax.ShapeDtypeStruct((NP, CH), _f32),
               jax.ShapeDtypeStruct((NP,), _f32),
               jax.ShapeDtypeStruct((NP,), _f32),
               jax.ShapeDtypeStruct((NP,), _f32)),
)

_tc_fin = pl.pallas_call(
    _tc_fin_body,
    out_shape=jax.ShapeDtypeStruct((NP, OUT), _f32),
)


def _sc_layer_body(h_hbm, as_hbm, ad_hbm, m_hbm, idx_hbm,
                   outp_hbm, denp_hbm,
                   rows_b, w_b, ase_b, ade_b, me_b, idx_b,
                   rows_sem, sc_sem, idx_sem, scat_sem,
                   out_sp, den_sp):
    c = lax.axis_index("c")
    s = lax.axis_index("s")
    wid = c * NS + s

    def _idx_copy(k):
        return pltpu.make_async_copy(
            idx_hbm.at[wid, k], idx_b.at[lax.rem(k, IRING)], idx_sem)

    def _gather_descs(k, r):
        k8 = lax.rem(k, IRING)
        return [
            pltpu.make_async_copy(h_hbm.at[idx_b.at[k8, 0]], rows_b.at[r],
                                  rows_sem),
            pltpu.make_async_copy(as_hbm.at[idx_b.at[k8, 0]], ase_b.at[r],
                                  sc_sem),
            pltpu.make_async_copy(ad_hbm.at[idx_b.at[k8, 1]], ade_b.at[r],
                                  sc_sem),
            pltpu.make_async_copy(m_hbm.at[idx_b.at[k8, 1]], me_b.at[r],
                                  sc_sem),
        ]

    def _issue_scatters(k, r):
        k8 = lax.rem(k, IRING)
        pltpu.async_copy(w_b.at[r], den_sp.at[idx_b.at[k8, 1]], scat_sem,
                         add=True)
        pltpu.async_copy(rows_b.at[r], out_sp.at[idx_b.at[k8, 1]], scat_sem,
                         add=True)

    def _wait_scatters(k, r):
        k8 = lax.rem(k, IRING)
        pltpu.make_async_copy(w_b.at[r], den_sp.at[idx_b.at[k8, 1]],
                              scat_sem).wait()
        pltpu.make_async_copy(rows_b.at[r], out_sp.at[idx_b.at[k8, 1]],
                              scat_sem).wait()

    zeros16 = jnp.zeros((16,), _f32)

    @pl.loop(0, BLK)
    def _zero_rows(i):
        for cg in range(CH // 16):
            rows_b[0, i, pl.ds(cg * 16, 16)] = zeros16

    for g in range(BLK // 16):
        w_b[0, pl.ds(g * 16, 16)] = zeros16
    for t in range(RPT // BLK):
        pltpu.sync_copy(rows_b.at[0], out_sp.at[pl.ds(s * RPT + t * BLK, BLK)])
        pltpu.sync_copy(w_b.at[0], den_sp.at[pl.ds(s * RPT + t * BLK, BLK)])
    _rem = RPT - (RPT // BLK) * BLK
    if _rem:
        pltpu.sync_copy(rows_b.at[0].at[pl.ds(0, _rem)],
                        out_sp.at[pl.ds(s * RPT + (RPT // BLK) * BLK, _rem)])
        pltpu.sync_copy(w_b.at[0].at[pl.ds(0, _rem)],
                        den_sp.at[pl.ds(s * RPT + (RPT // BLK) * BLK, _rem)])
    plsc.subcore_barrier()

    def _step(k, r):
        for d in _gather_descs(k, r):
            d.wait()
        for g in range(BLK // 16):
            a_s = ase_b[r, pl.ds(g * 16, 16)]
            a_d = ade_b[r, pl.ds(g * 16, 16)]
            mm = me_b[r, pl.ds(g * 16, 16)]
            t = a_s + a_d
            e = jnp.where(t > 0, t, 0.2 * t)
            w_b[r, pl.ds(g * 16, 16)] = jnp.exp(e - mm)

        @pl.loop(0, BLK, unroll=4)
        def _scale(i):
            bidx = jnp.zeros((16,), jnp.int32) + i
            a16 = plsc.load_gather(w_b.at[r], [bidx])
            for cg in range(CH // 16):
                rows_b[r, i, pl.ds(cg * 16, 16)] = (
                    rows_b[r, i, pl.ds(cg * 16, 16)] * a16)

        _issue_scatters(k, r)

        @pl.when(k >= 1)
        def _drain():
            _wait_scatters(k - 1, (r - 1) % NRING)

        @pl.when(k + 2 < NBLK)
        def _prefetch():
            _idx_copy(k + 2).wait()
            for d in _gather_descs(k + 2, (r + 2) % NRING):
                d.start()

        @pl.when(k + 4 < NBLK)
        def _prefetch_idx():
            _idx_copy(k + 4).start()

    for k in range(4):
        _idx_copy(k).start()
    for k in range(2):
        _idx_copy(k).wait()
        for d in _gather_descs(k, k):
            d.start()

    @pl.loop(0, NBLK // NRING)
    def _outer(t):
        for b in range(NRING):
            _step(t * NRING + b, b)

    _wait_scatters(NBLK - 1, (NBLK - 1) % NRING)

    plsc.subcore_barrier()
    pltpu.sync_copy(out_sp.at[pl.ds(s * RPT, RPT)],
                    outp_hbm.at[c, pl.ds(s * RPT, RPT)])
    pltpu.sync_copy(den_sp.at[pl.ds(s * RPT, RPT)],
                    denp_hbm.at[c, pl.ds(s * RPT, RPT)])


@functools.cache
def _make_sc_layer():
    mesh = plsc.VectorSubcoreMesh(core_axis_name="c", subcore_axis_name="s",
                                  num_cores=NC, num_subcores=NS)
    return pl.kernel(
        _sc_layer_body,
        out_type=(jax.ShapeDtypeStruct((NC, NP, CH), _f32),
                  jax.ShapeDtypeStruct((NC, NP), _f32)),
        mesh=mesh,
        compiler_params=pltpu.CompilerParams(needs_layout_passes=False),
        scratch_types=[
            pltpu.VMEM((NRING, BLK, CH), _f32),
            pltpu.VMEM((NRING, BLK), _f32),
            pltpu.VMEM((NRING, BLK), _f32),
            pltpu.VMEM((NRING, BLK), _f32),
            pltpu.VMEM((NRING, BLK), _f32),
            pltpu.VMEM((IRING, 2, BLK), jnp.int32),
            pltpu.SemaphoreType.DMA,
            pltpu.SemaphoreType.DMA,
            pltpu.SemaphoreType.DMA,
            pltpu.SemaphoreType.DMA,
            pltpu.VMEM_SHARED((NP, CH), _f32),
            pltpu.VMEM_SHARED((NP,), _f32),
        ],
    )


def kernel(x, edge_index, W0, a_src0, a_dst0, b0, W1, a_src1, a_dst1, b1,
           Wc, bc):
    ei = edge_index.astype(jnp.int32)
    ar = jnp.arange(N, dtype=jnp.int32)
    pad = jnp.full((EP - E_TOT,), PAD_IDX, jnp.int32)
    src = jnp.concatenate([ei[0], ar, pad]).reshape(NW, NBLK, BLK)
    dst = jnp.concatenate([ei[1], ar, pad]).reshape(NW, NBLK, BLK)
    idx = jnp.stack([src, dst], axis=2)
    xp = jnp.pad(x, ((0, NP - N), (0, 0)))

    sc_layer = _make_sc_layer()
    h0, as0, ad0, m0 = _tc_pre(xp, W0, a_src0, a_dst0)
    op0, dp0 = sc_layer(h0, as0, ad0, m0, idx)
    h1, as1, ad1, m1 = _tc_mid(op0, dp0, b0, W1, a_src1, a_dst1)
    op1, dp1 = sc_layer(h1, as1, ad1, m1, idx)
    y = _tc_fin(op1, dp1, b1, Wc, bc)
    return y[:N]
